# baseline (speedup 1.0000x reference)
import jax
import jax.numpy as jnp
from jax import lax
from jax.experimental import pallas as pl
from jax.experimental.pallas import tpu as pltpu
from jax.experimental.pallas import tpu_sc as plsc

N = 10000
E = 320000
D = 128
DE = 16
NG = 64
DZ = 32

NC = 2
NS = 16
NT = NC * NS
CS = 64
CH = 162
EP = NT * CH * CS

NP = 10240
RPT = NP // NS
BLK = 512
NBLK = NP // BLK

_f32 = jnp.float32
import functools


@functools.lru_cache(maxsize=None)
def _mesh():
    return plsc.VectorSubcoreMesh(core_axis_name="c", subcore_axis_name="s",
                                  num_cores=NC, num_subcores=NS)


NBUF = 3
GRP = CH // NBUF


def _sc_passa_body(x_hbm, srcb_hbm, dstb_hbm, zA_hbm,
                   outA_hbm,
                   sb0, sb1, sb2, db0, db1, db2, rb0, rb1, rb2,
                   i0, i1, i2, g0, g1, g2, t0, t1, t2, accA):
    sbufs = (sb0, sb1, sb2)
    dbufs = (db0, db1, db2)
    rbufs = (rb0, rb1, rb2)
    isems = (i0, i1, i2)
    gsems = (g0, g1, g2)
    ssems = (t0, t1, t2)
    c = lax.axis_index("c")
    s = lax.axis_index("s")
    wid = s * NC + c
    ebase = wid * CH * CS
    pltpu.sync_copy(zA_hbm, accA.at[pl.ds(s * RPT, RPT)])
    plsc.subcore_barrier()

    def fetch_idx(j, b):
        pltpu.async_copy(srcb_hbm.at[pl.ds(ebase + j * CS, CS)],
                         sbufs[b], isems[b])
        pltpu.async_copy(dstb_hbm.at[pl.ds(ebase + j * CS, CS)],
                         dbufs[b], isems[b])

    def wait_idx(j, b):
        pltpu.make_async_copy(srcb_hbm.at[pl.ds(ebase + j * CS, CS)],
                              sbufs[b], isems[b]).wait()
        pltpu.make_async_copy(dstb_hbm.at[pl.ds(ebase + j * CS, CS)],
                              dbufs[b], isems[b]).wait()

    for b in range(NBUF):
        fetch_idx(b, b)

    def group(g, carry):
        for b in range(NBUF):
            j = g * NBUF + b
            wait_idx(j, b)
            pltpu.async_copy(x_hbm.at[sbufs[b]], rbufs[b], gsems[b])
        for b in range(NBUF):
            pltpu.make_async_copy(x_hbm.at[sbufs[b]], rbufs[b],
                                  gsems[b]).wait()
            pltpu.async_copy(rbufs[b], accA.at[dbufs[b]], ssems[b], add=True)
        for b in range(NBUF):
            jn = g * NBUF + b + NBUF
            pltpu.make_async_copy(rbufs[b], accA.at[dbufs[b]],
                                  ssems[b]).wait()

            @pl.when(jn < CH)
            def _():
                fetch_idx(jn, b)

        return carry

    lax.fori_loop(0, GRP, group, 0)
    plsc.subcore_barrier()
    pltpu.sync_copy(accA.at[pl.ds(s * RPT, RPT)],
                    outA_hbm.at[pl.ds(c * NP + s * RPT, RPT)])


def _sc_passa(*args):
    return pl.kernel(
        _sc_passa_body,
        out_type=jax.ShapeDtypeStruct((NC * NP, D), _f32),
        mesh=_mesh(),
        scratch_types=[pltpu.VMEM((CS,), jnp.int32) for _ in range(2 * NBUF)]
          + [pltpu.VMEM((CS, D), _f32) for _ in range(NBUF)]
          + [pltpu.SemaphoreType.DMA for _ in range(3 * NBUF)]
          + [pltpu.VMEM_SHARED((NP, D), _f32)],
    )(*args)


def _sc_passz_body(ea_hbm, dstb_hbm, zZ_hbm,
                   outZ_hbm,
                   db0, db1, db2, eb0, eb1, eb2, wb0, wb1, wb2,
                   i0, i1, i2, t0, t1, t2, accZ):
    dbufs = (db0, db1, db2)
    ebufs = (eb0, eb1, eb2)
    wbufs = (wb0, wb1, wb2)
    isems = (i0, i1, i2)
    ssems = (t0, t1, t2)
    c = lax.axis_index("c")
    s = lax.axis_index("s")
    wid = s * NC + c
    ebase = wid * CH * CS
    pltpu.sync_copy(zZ_hbm, accZ.at[pl.ds(s * RPT, RPT)])
    for b in range(NBUF):
        pltpu.sync_copy(zZ_hbm.at[pl.ds(0, CS)], wbufs[b])
    plsc.subcore_barrier()

    def fetch(j, b):
        pltpu.async_copy(dstb_hbm.at[pl.ds(ebase + j * CS, CS)],
                         dbufs[b], isems[b])
        pltpu.async_copy(ea_hbm.at[pl.ds((ebase + j * CS) * DZ, CS * DZ)],
                         ebufs[b], isems[b])

    def wait_fetch(j, b):
        pltpu.make_async_copy(dstb_hbm.at[pl.ds(ebase + j * CS, CS)],
                              dbufs[b], isems[b]).wait()
        pltpu.make_async_copy(ea_hbm.at[pl.ds((ebase + j * CS) * DZ, CS * DZ)],
                              ebufs[b], isems[b]).wait()

    for b in range(NBUF):
        fetch(b, b)

    def group(g, carry):
        for b in range(NBUF):
            j = g * NBUF + b
            wait_fetch(j, b)

            def cp(r, carry2):
                wbufs[b][r, 0:16] = ebufs[b][pl.ds(r * DZ, 16)]
                wbufs[b][r, 16:32] = ebufs[b][pl.ds(r * DZ + 16, 16)]
                return carry2

            lax.fori_loop(0, CS, cp, 0)
            pltpu.async_copy(wbufs[b], accZ.at[dbufs[b]], ssems[b], add=True)
        for b in range(NBUF):
            jn = g * NBUF + b + NBUF
            pltpu.make_async_copy(wbufs[b], accZ.at[dbufs[b]],
                                  ssems[b]).wait()

            @pl.when(jn < CH)
            def _():
                fetch(jn, b)

        return carry

    lax.fori_loop(0, GRP, group, 0)
    plsc.subcore_barrier()
    pltpu.sync_copy(accZ.at[pl.ds(s * RPT, RPT)],
                    outZ_hbm.at[pl.ds(c * NP + s * RPT, RPT)])


def _sc_passz(*args):
    return pl.kernel(
        _sc_passz_body,
        out_type=jax.ShapeDtypeStruct((NC * NP, D), _f32),
        mesh=_mesh(),
        scratch_types=[pltpu.VMEM((CS,), jnp.int32) for _ in range(NBUF)]
          + [pltpu.VMEM((CS * DZ,), _f32) for _ in range(NBUF)]
          + [pltpu.VMEM((CS, D), _f32) for _ in range(NBUF)]
          + [pltpu.SemaphoreType.DMA for _ in range(2 * NBUF)]
          + [pltpu.VMEM_SHARED((NP, D), _f32)],
    )(*args)


def _wprep_body(wm_ref, bm_ref, wl_ref, wa_ref, wz_ref):
    wl = wl_ref[...]
    wa_ref[...] = jnp.dot(wm_ref[0:D, :], wl, preferred_element_type=_f32)
    wz_ref[0:DE, :] = jnp.dot(wm_ref[D:D + DE, :], wl,
                              preferred_element_type=_f32)
    bmw = jnp.dot(bm_ref[...], wl, preferred_element_type=_f32)
    wz_ref[DE:DZ, :] = jnp.broadcast_to(bmw, (DZ - DE, D)) / (DZ - DE)
    wz_ref[DZ:D, :] = jnp.zeros((D - DZ, D), _f32)


def _wprep(wm, bm, wl):
    return pl.pallas_call(
        _wprep_body,
        out_shape=[jax.ShapeDtypeStruct((D, D), _f32),
                   jax.ShapeDtypeStruct((D, D), _f32)],
    )(wm, bm.reshape(1, D), wl)


def _dense_body(a0, a1, xin, z0, z1, wa, wr, wz, b, h_ref):
    a = a0[...] + a1[...]
    z = z0[...] + z1[...]
    acc = jnp.dot(a, wa[...], preferred_element_type=_f32)
    acc = acc + jnp.dot(xin[...], wr[...], preferred_element_type=_f32)
    acc = acc + jnp.dot(z, wz[...], preferred_element_type=_f32)
    h_ref[...] = jnp.maximum(acc + b[...], 0.0)


def _dense(A, xin, Z, wa, wr, wz, b):
    return pl.pallas_call(
        _dense_body,
        grid=(NBLK,),
        in_specs=[
            pl.BlockSpec((BLK, D), lambda i: (i, 0)),
            pl.BlockSpec((BLK, D), lambda i: (NBLK + i, 0)),
            pl.BlockSpec((BLK, D), lambda i: (i, 0)),
            pl.BlockSpec((BLK, D), lambda i: (i, 0)),
            pl.BlockSpec((BLK, D), lambda i: (NBLK + i, 0)),
            pl.BlockSpec((D, D), lambda i: (0, 0)),
            pl.BlockSpec((D, D), lambda i: (0, 0)),
            pl.BlockSpec((D, D), lambda i: (0, 0)),
            pl.BlockSpec((1, D), lambda i: (0, 0)),
        ],
        out_specs=pl.BlockSpec((BLK, D), lambda i: (i, 0)),
        out_shape=jax.ShapeDtypeStruct((NP, D), _f32),
    )(A, A, xin, Z, Z, wa, wr, wz, b.reshape(1, D))


def _dense2_body(a0, a1, xin, z0, z1, wa, wr, wz, b, bat, out_ref):
    i = pl.program_id(0)
    a = a0[...] + a1[...]
    z = z0[...] + z1[...]
    acc = jnp.dot(a, wa[...], preferred_element_type=_f32)
    acc = acc + jnp.dot(xin[...], wr[...], preferred_element_type=_f32)
    acc = acc + jnp.dot(z, wz[...], preferred_element_type=_f32)
    h2 = jnp.maximum(acc + b[...], 0.0)
    seg = bat[0]
    iota = lax.broadcasted_iota(jnp.int32, (NG, BLK), 0)
    oh = jnp.where(jnp.broadcast_to(seg, (NG, BLK)) == iota, 1.0, 0.0)
    contrib = jnp.dot(oh, h2, preferred_element_type=_f32)

    @pl.when(i == 0)
    def _():
        out_ref[...] = contrib

    @pl.when(i != 0)
    def _():
        out_ref[...] += contrib


def _dense2(A, h1, Z, wa, wr, wz, b, batch3):
    return pl.pallas_call(
        _dense2_body,
        grid=(NBLK,),
        in_specs=[
            pl.BlockSpec((BLK, D), lambda i: (i, 0)),
            pl.BlockSpec((BLK, D), lambda i: (NBLK + i, 0)),
            pl.BlockSpec((BLK, D), lambda i: (i, 0)),
            pl.BlockSpec((BLK, D), lambda i: (i, 0)),
            pl.BlockSpec((BLK, D), lambda i: (NBLK + i, 0)),
            pl.BlockSpec((D, D), lambda i: (0, 0)),
            pl.BlockSpec((D, D), lambda i: (0, 0)),
            pl.BlockSpec((D, D), lambda i: (0, 0)),
            pl.BlockSpec((1, D), lambda i: (0, 0)),
            pl.BlockSpec((1, 1, BLK), lambda i: (i, 0, 0)),
        ],
        out_specs=pl.BlockSpec((NG, D), lambda i: (0, 0)),
        out_shape=jax.ShapeDtypeStruct((NG, D), _f32),
    )(A, A, h1, Z, Z, wa, wr, wz, b.reshape(1, D), batch3)


def kernel(x, edge_index, edge_attr, batch,
           Wm0, bm0, Wl0, Wr0, b0,
           Wm1, bm1, Wl1, Wr1, b1):
    src = edge_index[0]
    dst = edge_index[1]
    pad_e = EP - E
    src_p = jnp.concatenate([src, jnp.zeros((pad_e,), jnp.int32)])
    dst_p = jnp.concatenate([dst, jnp.full((pad_e,), N, jnp.int32)])
    ea_p = jnp.concatenate(
        [edge_attr, jnp.ones((E, DZ - DE), _f32)], axis=1)
    ea_p = jnp.concatenate([ea_p, jnp.zeros((pad_e, DZ), _f32)], axis=0)
    x_p = jnp.concatenate([x, jnp.zeros((NP - N, D), _f32)], axis=0)
    batch3 = jnp.concatenate(
        [batch, jnp.full((NP - N,), NG, jnp.int32)]).reshape(NBLK, 1, BLK)
    zA = jnp.zeros((RPT, D), _f32)

    wa0, wz0 = _wprep(Wm0, bm0, Wl0)
    wa1, wz1 = _wprep(Wm1, bm1, Wl1)

    Zp = _sc_passz(ea_p.reshape(-1), dst_p, zA)
    A0 = _sc_passa(x_p, src_p, dst_p, zA)
    h1 = _dense(A0, x_p, Zp, wa0, Wr0, wz0, b0)
    A1 = _sc_passa(h1, src_p, dst_p, zA)
    out = _dense2(A1, h1, Zp, wa1, Wr1, wz1, b1, batch3)
    return out


# speedup vs baseline: 1.0715x; 1.0715x over previous
import jax
import jax.numpy as jnp
from jax import lax
from jax.experimental import pallas as pl
from jax.experimental.pallas import tpu as pltpu
from jax.experimental.pallas import tpu_sc as plsc

N = 10000
E = 320000
D = 128
DE = 16
NG = 64
DZ = 32

NC = 2
NS = 16
NT = NC * NS
CS = 64
CH = 162
CSA = 120
CHA = 86
NBA = 2
GRPA = CHA // NBA
EP = NT * CH * CS

NP = 10240
RPT = NP // NS
BLK = 512
NBLK = NP // BLK

_f32 = jnp.float32
import functools


@functools.lru_cache(maxsize=None)
def _mesh():
    return plsc.VectorSubcoreMesh(core_axis_name="c", subcore_axis_name="s",
                                  num_cores=NC, num_subcores=NS)


NBUF = 3
GRP = CH // NBUF


def _sc_passa_body(x_hbm, srcb_hbm, dstb_hbm, zA_hbm,
                   outA_hbm,
                   sb0, sb1, db0, db1, rb0, rb1,
                   i0, i1, g0, g1, t0, t1, accA):
    sbufs = (sb0, sb1)
    dbufs = (db0, db1)
    rbufs = (rb0, rb1)
    isems = (i0, i1)
    gsems = (g0, g1)
    ssems = (t0, t1)
    c = lax.axis_index("c")
    s = lax.axis_index("s")
    wid = s * NC + c
    ebase = wid * CHA * CSA
    pltpu.sync_copy(zA_hbm, accA.at[pl.ds(s * RPT, RPT)])
    plsc.subcore_barrier()

    def fetch_idx(j, b):
        pltpu.async_copy(srcb_hbm.at[pl.ds(ebase + j * CSA, CSA)],
                         sbufs[b], isems[b])
        pltpu.async_copy(dstb_hbm.at[pl.ds(ebase + j * CSA, CSA)],
                         dbufs[b], isems[b])

    def wait_idx(j, b):
        pltpu.make_async_copy(srcb_hbm.at[pl.ds(ebase + j * CSA, CSA)],
                              sbufs[b], isems[b]).wait()
        pltpu.make_async_copy(dstb_hbm.at[pl.ds(ebase + j * CSA, CSA)],
                              dbufs[b], isems[b]).wait()

    for b in range(NBA):
        fetch_idx(b, b)

    def group(g, carry):
        for b in range(NBA):
            j = g * NBA + b
            wait_idx(j, b)
            pltpu.async_copy(x_hbm.at[sbufs[b]], rbufs[b], gsems[b])
        for b in range(NBA):
            pltpu.make_async_copy(x_hbm.at[sbufs[b]], rbufs[b],
                                  gsems[b]).wait()
            pltpu.async_copy(rbufs[b], accA.at[dbufs[b]], ssems[b], add=True)
        for b in range(NBA):
            jn = g * NBA + b + NBA
            pltpu.make_async_copy(rbufs[b], accA.at[dbufs[b]],
                                  ssems[b]).wait()

            @pl.when(jn < CHA)
            def _():
                fetch_idx(jn, b)

        return carry

    lax.fori_loop(0, GRPA, group, 0)
    plsc.subcore_barrier()
    pltpu.sync_copy(accA.at[pl.ds(s * RPT, RPT)],
                    outA_hbm.at[pl.ds(c * NP + s * RPT, RPT)])


def _sc_passa(*args):
    return pl.kernel(
        _sc_passa_body,
        out_type=jax.ShapeDtypeStruct((NC * NP, D), _f32),
        mesh=_mesh(),
        scratch_types=[pltpu.VMEM((CSA,), jnp.int32) for _ in range(2 * NBA)]
          + [pltpu.VMEM((CSA, D), _f32) for _ in range(NBA)]
          + [pltpu.SemaphoreType.DMA for _ in range(3 * NBA)]
          + [pltpu.VMEM_SHARED((NP, D), _f32)],
    )(*args)


def _sc_passz_body(ea_hbm, dstb_hbm, zZ_hbm,
                   outZ_hbm,
                   db0, db1, db2, eb0, eb1, eb2, wb0, wb1, wb2,
                   i0, i1, i2, t0, t1, t2, accZ):
    dbufs = (db0, db1, db2)
    ebufs = (eb0, eb1, eb2)
    wbufs = (wb0, wb1, wb2)
    isems = (i0, i1, i2)
    ssems = (t0, t1, t2)
    c = lax.axis_index("c")
    s = lax.axis_index("s")
    wid = s * NC + c
    ebase = wid * CH * CS
    pltpu.sync_copy(zZ_hbm, accZ.at[pl.ds(s * RPT, RPT)])
    for b in range(NBUF):
        pltpu.sync_copy(zZ_hbm.at[pl.ds(0, CS)], wbufs[b])
    plsc.subcore_barrier()

    def fetch(j, b):
        pltpu.async_copy(dstb_hbm.at[pl.ds(ebase + j * CS, CS)],
                         dbufs[b], isems[b])
        pltpu.async_copy(ea_hbm.at[pl.ds((ebase + j * CS) * DZ, CS * DZ)],
                         ebufs[b], isems[b])

    def wait_fetch(j, b):
        pltpu.make_async_copy(dstb_hbm.at[pl.ds(ebase + j * CS, CS)],
                              dbufs[b], isems[b]).wait()
        pltpu.make_async_copy(ea_hbm.at[pl.ds((ebase + j * CS) * DZ, CS * DZ)],
                              ebufs[b], isems[b]).wait()

    for b in range(NBUF):
        fetch(b, b)

    def group(g, carry):
        for b in range(NBUF):
            j = g * NBUF + b
            wait_fetch(j, b)

            def cp(r, carry2):
                wbufs[b][r, 0:16] = ebufs[b][pl.ds(r * DZ, 16)]
                wbufs[b][r, 16:32] = ebufs[b][pl.ds(r * DZ + 16, 16)]
                return carry2

            lax.fori_loop(0, CS, cp, 0)
            pltpu.async_copy(wbufs[b], accZ.at[dbufs[b]], ssems[b], add=True)
        for b in range(NBUF):
            jn = g * NBUF + b + NBUF
            pltpu.make_async_copy(wbufs[b], accZ.at[dbufs[b]],
                                  ssems[b]).wait()

            @pl.when(jn < CH)
            def _():
                fetch(jn, b)

        return carry

    lax.fori_loop(0, GRP, group, 0)
    plsc.subcore_barrier()
    pltpu.sync_copy(accZ.at[pl.ds(s * RPT, RPT)],
                    outZ_hbm.at[pl.ds(c * NP + s * RPT, RPT)])


def _sc_passz(*args):
    return pl.kernel(
        _sc_passz_body,
        out_type=jax.ShapeDtypeStruct((NC * NP, D), _f32),
        mesh=_mesh(),
        scratch_types=[pltpu.VMEM((CS,), jnp.int32) for _ in range(NBUF)]
          + [pltpu.VMEM((CS * DZ,), _f32) for _ in range(NBUF)]
          + [pltpu.VMEM((CS, D), _f32) for _ in range(NBUF)]
          + [pltpu.SemaphoreType.DMA for _ in range(2 * NBUF)]
          + [pltpu.VMEM_SHARED((NP, D), _f32)],
    )(*args)


def _wprep_body(wm_ref, bm_ref, wl_ref, wa_ref, wz_ref):
    wl = wl_ref[...]
    wa_ref[...] = jnp.dot(wm_ref[0:D, :], wl, preferred_element_type=_f32)
    wz_ref[0:DE, :] = jnp.dot(wm_ref[D:D + DE, :], wl,
                              preferred_element_type=_f32)
    bmw = jnp.dot(bm_ref[...], wl, preferred_element_type=_f32)
    wz_ref[DE:DZ, :] = jnp.broadcast_to(bmw, (DZ - DE, D)) / (DZ - DE)
    wz_ref[DZ:D, :] = jnp.zeros((D - DZ, D), _f32)


def _wprep(wm, bm, wl):
    return pl.pallas_call(
        _wprep_body,
        out_shape=[jax.ShapeDtypeStruct((D, D), _f32),
                   jax.ShapeDtypeStruct((D, D), _f32)],
    )(wm, bm.reshape(1, D), wl)


def _dense_body(a0, a1, xin, z0, z1, wa, wr, wz, b, h_ref):
    a = a0[...] + a1[...]
    z = z0[...] + z1[...]
    acc = jnp.dot(a, wa[...], preferred_element_type=_f32)
    acc = acc + jnp.dot(xin[...], wr[...], preferred_element_type=_f32)
    acc = acc + jnp.dot(z, wz[...], preferred_element_type=_f32)
    h_ref[...] = jnp.maximum(acc + b[...], 0.0)


def _dense(A, xin, Z, wa, wr, wz, b):
    return pl.pallas_call(
        _dense_body,
        grid=(NBLK,),
        in_specs=[
            pl.BlockSpec((BLK, D), lambda i: (i, 0)),
            pl.BlockSpec((BLK, D), lambda i: (NBLK + i, 0)),
            pl.BlockSpec((BLK, D), lambda i: (i, 0)),
            pl.BlockSpec((BLK, D), lambda i: (i, 0)),
            pl.BlockSpec((BLK, D), lambda i: (NBLK + i, 0)),
            pl.BlockSpec((D, D), lambda i: (0, 0)),
            pl.BlockSpec((D, D), lambda i: (0, 0)),
            pl.BlockSpec((D, D), lambda i: (0, 0)),
            pl.BlockSpec((1, D), lambda i: (0, 0)),
        ],
        out_specs=pl.BlockSpec((BLK, D), lambda i: (i, 0)),
        out_shape=jax.ShapeDtypeStruct((NP, D), _f32),
    )(A, A, xin, Z, Z, wa, wr, wz, b.reshape(1, D))


def _dense2_body(a0, a1, xin, z0, z1, wa, wr, wz, b, bat, out_ref):
    i = pl.program_id(0)
    a = a0[...] + a1[...]
    z = z0[...] + z1[...]
    acc = jnp.dot(a, wa[...], preferred_element_type=_f32)
    acc = acc + jnp.dot(xin[...], wr[...], preferred_element_type=_f32)
    acc = acc + jnp.dot(z, wz[...], preferred_element_type=_f32)
    h2 = jnp.maximum(acc + b[...], 0.0)
    seg = bat[0]
    iota = lax.broadcasted_iota(jnp.int32, (NG, BLK), 0)
    oh = jnp.where(jnp.broadcast_to(seg, (NG, BLK)) == iota, 1.0, 0.0)
    contrib = jnp.dot(oh, h2, preferred_element_type=_f32)

    @pl.when(i == 0)
    def _():
        out_ref[...] = contrib

    @pl.when(i != 0)
    def _():
        out_ref[...] += contrib


def _dense2(A, h1, Z, wa, wr, wz, b, batch3):
    return pl.pallas_call(
        _dense2_body,
        grid=(NBLK,),
        in_specs=[
            pl.BlockSpec((BLK, D), lambda i: (i, 0)),
            pl.BlockSpec((BLK, D), lambda i: (NBLK + i, 0)),
            pl.BlockSpec((BLK, D), lambda i: (i, 0)),
            pl.BlockSpec((BLK, D), lambda i: (i, 0)),
            pl.BlockSpec((BLK, D), lambda i: (NBLK + i, 0)),
            pl.BlockSpec((D, D), lambda i: (0, 0)),
            pl.BlockSpec((D, D), lambda i: (0, 0)),
            pl.BlockSpec((D, D), lambda i: (0, 0)),
            pl.BlockSpec((1, D), lambda i: (0, 0)),
            pl.BlockSpec((1, 1, BLK), lambda i: (i, 0, 0)),
        ],
        out_specs=pl.BlockSpec((NG, D), lambda i: (0, 0)),
        out_shape=jax.ShapeDtypeStruct((NG, D), _f32),
    )(A, A, h1, Z, Z, wa, wr, wz, b.reshape(1, D), batch3)


def kernel(x, edge_index, edge_attr, batch,
           Wm0, bm0, Wl0, Wr0, b0,
           Wm1, bm1, Wl1, Wr1, b1):
    src = edge_index[0]
    dst = edge_index[1]
    pad_e = EP - E
    src_p = jnp.concatenate([src, jnp.zeros((pad_e,), jnp.int32)])
    dst_p = jnp.concatenate([dst, jnp.full((pad_e,), N, jnp.int32)])
    ea_p = jnp.concatenate(
        [edge_attr, jnp.ones((E, DZ - DE), _f32)], axis=1)
    ea_p = jnp.concatenate([ea_p, jnp.zeros((pad_e, DZ), _f32)], axis=0)
    x_p = jnp.concatenate([x, jnp.zeros((NP - N, D), _f32)], axis=0)
    batch3 = jnp.concatenate(
        [batch, jnp.full((NP - N,), NG, jnp.int32)]).reshape(NBLK, 1, BLK)
    zA = jnp.zeros((RPT, D), _f32)

    wa0, wz0 = _wprep(Wm0, bm0, Wl0)
    wa1, wz1 = _wprep(Wm1, bm1, Wl1)

    Zp = _sc_passz(ea_p.reshape(-1), dst_p, zA)
    A0 = _sc_passa(x_p, src_p, dst_p, zA)
    h1 = _dense(A0, x_p, Zp, wa0, Wr0, wz0, b0)
    A1 = _sc_passa(h1, src_p, dst_p, zA)
    out = _dense2(A1, h1, Zp, wa1, Wr1, wz1, b1, batch3)
    return out


# speedup vs baseline: 1.8957x; 1.7693x over previous
import jax
import jax.numpy as jnp
from jax import lax
from jax.experimental import pallas as pl
from jax.experimental.pallas import tpu as pltpu
from jax.experimental.pallas import tpu_sc as plsc

N = 10000
E = 320000
D = 128
DE = 16
NG = 64
DZ = 32

NC = 2
NS = 16
NT = NC * NS
CS = 128
CH = 80
EP = NT * CH * CS

NP = 10240
RPT = NP // NS
BLK = 512
NBLK = NP // BLK

_f32 = jnp.float32
import functools


@functools.lru_cache(maxsize=None)
def _mesh():
    return plsc.VectorSubcoreMesh(core_axis_name="c", subcore_axis_name="s",
                                  num_cores=NC, num_subcores=NS)


def _sc_passa_body(x_hbm, srcb_hbm, dstb_hbm, zA_hbm,
                   outA_hbm,
                   src_cur, dst_cur, rows_v, accA, sem):
    c = lax.axis_index("c")
    s = lax.axis_index("s")
    wid = s * NC + c
    nch = jnp.minimum(CH, (E - wid * CH * CS + CS - 1) // CS)
    pltpu.sync_copy(zA_hbm, accA.at[pl.ds(s * RPT, RPT)])
    plsc.subcore_barrier()

    def body(j, carry):
        base = (wid * CH + j) * CS
        pltpu.sync_copy(srcb_hbm.at[pl.ds(base, CS)], src_cur)
        pltpu.sync_copy(dstb_hbm.at[pl.ds(base, CS)], dst_cur)
        pltpu.async_copy(x_hbm.at[src_cur], rows_v, sem).wait()
        pltpu.sync_copy(rows_v, accA.at[dst_cur], add=True)
        return carry

    lax.fori_loop(0, nch, body, 0)
    plsc.subcore_barrier()
    pltpu.sync_copy(accA.at[pl.ds(s * RPT, RPT)],
                    outA_hbm.at[pl.ds(c * NP + s * RPT, RPT)])


def _sc_passa(*args):
    return pl.kernel(
        _sc_passa_body,
        out_type=jax.ShapeDtypeStruct((NC * NP, D), _f32),
        mesh=_mesh(),
        scratch_types=[
            pltpu.VMEM((CS,), jnp.int32),
            pltpu.VMEM((CS,), jnp.int32),
            pltpu.VMEM((CS, D), _f32),
            pltpu.VMEM_SHARED((NP, D), _f32),
            pltpu.SemaphoreType.DMA,
        ],
    )(*args)


def _sc_passz_body(ea_hbm, dstb_hbm, zZ_hbm,
                   outZ_hbm,
                   dst_cur, ea32_v, ea_v, accZ):
    c = lax.axis_index("c")
    s = lax.axis_index("s")
    wid = s * NC + c
    nch = jnp.minimum(CH, (E - wid * CH * CS + CS - 1) // CS)
    pltpu.sync_copy(zZ_hbm, accZ.at[pl.ds(s * RPT, RPT)])
    pltpu.sync_copy(zZ_hbm.at[pl.ds(0, CS)], ea_v)
    plsc.subcore_barrier()

    def body(j, carry):
        base = (wid * CH + j) * CS
        pltpu.sync_copy(dstb_hbm.at[pl.ds(base, CS)], dst_cur)
        pltpu.sync_copy(ea_hbm.at[pl.ds(base, CS)], ea32_v)

        def cp(r, carry2):
            ea_v[r, 0:16] = ea32_v[r, 0:16]
            ea_v[r, 16:32] = ea32_v[r, 16:32]
            return carry2

        lax.fori_loop(0, CS, cp, 0)
        pltpu.sync_copy(ea_v, accZ.at[dst_cur], add=True)
        return carry

    lax.fori_loop(0, nch, body, 0)
    plsc.subcore_barrier()
    pltpu.sync_copy(accZ.at[pl.ds(s * RPT, RPT)],
                    outZ_hbm.at[pl.ds(c * NP + s * RPT, RPT)])


def _sc_passz(*args):
    return pl.kernel(
        _sc_passz_body,
        out_type=jax.ShapeDtypeStruct((NC * NP, D), _f32),
        mesh=_mesh(),
        scratch_types=[
            pltpu.VMEM((CS,), jnp.int32),
            pltpu.VMEM((CS, DZ), _f32),
            pltpu.VMEM((CS, D), _f32),
            pltpu.VMEM_SHARED((NP, D), _f32),
        ],
    )(*args)


def _wprep_body(wm_ref, bm_ref, wl_ref, wa_ref, wz_ref):
    wl = wl_ref[...]
    wa_ref[...] = jnp.dot(wm_ref[0:D, :], wl, preferred_element_type=_f32)
    wz_ref[0:DE, :] = jnp.dot(wm_ref[D:D + DE, :], wl,
                              preferred_element_type=_f32)
    bmw = jnp.dot(bm_ref[...], wl, preferred_element_type=_f32)
    wz_ref[DE:DZ, :] = jnp.broadcast_to(bmw, (DZ - DE, D)) / (DZ - DE)
    wz_ref[DZ:D, :] = jnp.zeros((D - DZ, D), _f32)


def _wprep(wm, bm, wl):
    return pl.pallas_call(
        _wprep_body,
        out_shape=[jax.ShapeDtypeStruct((D, D), _f32),
                   jax.ShapeDtypeStruct((D, D), _f32)],
    )(wm, bm.reshape(1, D), wl)


def _dense_body(a0, a1, xin, z0, z1, wa, wr, wz, b, h_ref):
    a = a0[...] + a1[...]
    z = z0[...] + z1[...]
    acc = jnp.dot(a, wa[...], preferred_element_type=_f32)
    acc = acc + jnp.dot(xin[...], wr[...], preferred_element_type=_f32)
    acc = acc + jnp.dot(z, wz[...], preferred_element_type=_f32)
    h_ref[...] = jnp.maximum(acc + b[...], 0.0)


def _dense(A, xin, Z, wa, wr, wz, b):
    return pl.pallas_call(
        _dense_body,
        grid=(NBLK,),
        in_specs=[
            pl.BlockSpec((BLK, D), lambda i: (i, 0)),
            pl.BlockSpec((BLK, D), lambda i: (NBLK + i, 0)),
            pl.BlockSpec((BLK, D), lambda i: (i, 0)),
            pl.BlockSpec((BLK, D), lambda i: (i, 0)),
            pl.BlockSpec((BLK, D), lambda i: (NBLK + i, 0)),
            pl.BlockSpec((D, D), lambda i: (0, 0)),
            pl.BlockSpec((D, D), lambda i: (0, 0)),
            pl.BlockSpec((D, D), lambda i: (0, 0)),
            pl.BlockSpec((1, D), lambda i: (0, 0)),
        ],
        out_specs=pl.BlockSpec((BLK, D), lambda i: (i, 0)),
        out_shape=jax.ShapeDtypeStruct((NP, D), _f32),
    )(A, A, xin, Z, Z, wa, wr, wz, b.reshape(1, D))


def _dense2_body(a0, a1, xin, z0, z1, wa, wr, wz, b, bat, out_ref):
    i = pl.program_id(0)
    a = a0[...] + a1[...]
    z = z0[...] + z1[...]
    acc = jnp.dot(a, wa[...], preferred_element_type=_f32)
    acc = acc + jnp.dot(xin[...], wr[...], preferred_element_type=_f32)
    acc = acc + jnp.dot(z, wz[...], preferred_element_type=_f32)
    h2 = jnp.maximum(acc + b[...], 0.0)
    seg = bat[0]
    iota = lax.broadcasted_iota(jnp.int32, (NG, BLK), 0)
    oh = jnp.where(jnp.broadcast_to(seg, (NG, BLK)) == iota, 1.0, 0.0)
    contrib = jnp.dot(oh, h2, preferred_element_type=_f32)

    @pl.when(i == 0)
    def _():
        out_ref[...] = contrib

    @pl.when(i != 0)
    def _():
        out_ref[...] += contrib


def _dense2(A, h1, Z, wa, wr, wz, b, batch3):
    return pl.pallas_call(
        _dense2_body,
        grid=(NBLK,),
        in_specs=[
            pl.BlockSpec((BLK, D), lambda i: (i, 0)),
            pl.BlockSpec((BLK, D), lambda i: (NBLK + i, 0)),
            pl.BlockSpec((BLK, D), lambda i: (i, 0)),
            pl.BlockSpec((BLK, D), lambda i: (i, 0)),
            pl.BlockSpec((BLK, D), lambda i: (NBLK + i, 0)),
            pl.BlockSpec((D, D), lambda i: (0, 0)),
            pl.BlockSpec((D, D), lambda i: (0, 0)),
            pl.BlockSpec((D, D), lambda i: (0, 0)),
            pl.BlockSpec((1, D), lambda i: (0, 0)),
            pl.BlockSpec((1, 1, BLK), lambda i: (i, 0, 0)),
        ],
        out_specs=pl.BlockSpec((NG, D), lambda i: (0, 0)),
        out_shape=jax.ShapeDtypeStruct((NG, D), _f32),
    )(A, A, h1, Z, Z, wa, wr, wz, b.reshape(1, D), batch3)


def kernel(x, edge_index, edge_attr, batch,
           Wm0, bm0, Wl0, Wr0, b0,
           Wm1, bm1, Wl1, Wr1, b1):
    src = edge_index[0]
    dst = edge_index[1]
    pad_e = EP - E
    src_p = jnp.concatenate([src, jnp.zeros((pad_e,), jnp.int32)])
    dst_p = jnp.concatenate([dst, jnp.full((pad_e,), N, jnp.int32)])
    ea_p = jnp.concatenate(
        [edge_attr, jnp.ones((E, DZ - DE), _f32)], axis=1)
    ea_p = jnp.concatenate([ea_p, jnp.zeros((pad_e, DZ), _f32)], axis=0)
    x_p = jnp.concatenate([x, jnp.zeros((NP - N, D), _f32)], axis=0)
    batch3 = jnp.concatenate(
        [batch, jnp.full((NP - N,), NG, jnp.int32)]).reshape(NBLK, 1, BLK)
    zA = jnp.zeros((RPT, D), _f32)

    wa0, wz0 = _wprep(Wm0, bm0, Wl0)
    wa1, wz1 = _wprep(Wm1, bm1, Wl1)

    Zp = _sc_passz(ea_p, dst_p, zA)
    A0 = _sc_passa(x_p, src_p, dst_p, zA)
    h1 = _dense(A0, x_p, Zp, wa0, Wr0, wz0, b0)
    A1 = _sc_passa(h1, src_p, dst_p, zA)
    out = _dense2(A1, h1, Zp, wa1, Wr1, wz1, b1, batch3)
    return out


# speedup vs baseline: 2.3501x; 1.2397x over previous
import jax
import jax.numpy as jnp
from jax import lax
from jax.experimental import pallas as pl
from jax.experimental.pallas import tpu as pltpu
from jax.experimental.pallas import tpu_sc as plsc

N = 10000
E = 320000
D = 128
DE = 16
NG = 64
DZ = 32

NC = 2
NS = 16
NT = NC * NS
CS = 128
CH = 80
CSA = 120
CHA = 86
NBA = 2
EP = NT * CHA * CSA

NP = 10240
RPT = NP // NS
BLK = 512
NBLK = NP // BLK

_f32 = jnp.float32
import functools


@functools.lru_cache(maxsize=None)
def _mesh():
    return plsc.VectorSubcoreMesh(core_axis_name="c", subcore_axis_name="s",
                                  num_cores=NC, num_subcores=NS)


def _sc_passa_body(x_hbm, srcb_hbm, dstb_hbm, zA_hbm,
                   outA_hbm,
                   sb0, sb1, db0, db1, rb0, rb1,
                   i0, i1, g0, g1, t0, t1, accA):
    sbufs = (sb0, sb1)
    dbufs = (db0, db1)
    rbufs = (rb0, rb1)
    isems = (i0, i1)
    gsems = (g0, g1)
    ssems = (t0, t1)
    c = lax.axis_index("c")
    s = lax.axis_index("s")
    wid = s * NC + c
    ebase = wid * CHA * CSA
    nch = jnp.minimum(CHA, (E - ebase + CSA - 1) // CSA)
    ngrp = (nch + NBA - 1) // NBA
    pltpu.sync_copy(zA_hbm, accA.at[pl.ds(s * RPT, RPT)])
    plsc.subcore_barrier()

    def fetch_idx(j, b):
        pltpu.async_copy(srcb_hbm.at[pl.ds(ebase + j * CSA, CSA)],
                         sbufs[b], isems[b])
        pltpu.async_copy(dstb_hbm.at[pl.ds(ebase + j * CSA, CSA)],
                         dbufs[b], isems[b])

    def wait_idx(j, b):
        pltpu.make_async_copy(srcb_hbm.at[pl.ds(ebase + j * CSA, CSA)],
                              sbufs[b], isems[b]).wait()
        pltpu.make_async_copy(dstb_hbm.at[pl.ds(ebase + j * CSA, CSA)],
                              dbufs[b], isems[b]).wait()

    for b in range(NBA):
        @pl.when(b < nch)
        def _():
            fetch_idx(b, b)

    def group(g, carry):
        for b in range(NBA):
            j = g * NBA + b

            @pl.when(j < nch)
            def _():
                wait_idx(j, b)
                pltpu.async_copy(x_hbm.at[sbufs[b]], rbufs[b], gsems[b])
        for b in range(NBA):
            j = g * NBA + b

            @pl.when(j < nch)
            def _():
                pltpu.make_async_copy(x_hbm.at[sbufs[b]], rbufs[b],
                                      gsems[b]).wait()
                pltpu.async_copy(rbufs[b], accA.at[dbufs[b]], ssems[b],
                                 add=True)
        for b in range(NBA):
            j = g * NBA + b
            jn = j + NBA

            @pl.when(j < nch)
            def _():
                pltpu.make_async_copy(rbufs[b], accA.at[dbufs[b]],
                                      ssems[b]).wait()

                @pl.when(jn < nch)
                def _():
                    fetch_idx(jn, b)

        return carry

    lax.fori_loop(0, ngrp, group, 0)
    plsc.subcore_barrier()
    pltpu.sync_copy(accA.at[pl.ds(s * RPT, RPT)],
                    outA_hbm.at[pl.ds(c * NP + s * RPT, RPT)])


def _sc_passa(*args):
    return pl.kernel(
        _sc_passa_body,
        out_type=jax.ShapeDtypeStruct((NC * NP, D), _f32),
        mesh=_mesh(),
        scratch_types=[pltpu.VMEM((CSA,), jnp.int32) for _ in range(2 * NBA)]
          + [pltpu.VMEM((CSA, D), _f32) for _ in range(NBA)]
          + [pltpu.SemaphoreType.DMA for _ in range(3 * NBA)]
          + [pltpu.VMEM_SHARED((NP, D), _f32)],
    )(*args)


def _sc_passz_body(ea_hbm, dstb_hbm, zZ_hbm,
                   outZ_hbm,
                   dst_cur, ea32_v, ea_v, accZ):
    c = lax.axis_index("c")
    s = lax.axis_index("s")
    wid = s * NC + c
    nch = jnp.minimum(CH, (E - wid * CH * CS + CS - 1) // CS)
    pltpu.sync_copy(zZ_hbm, accZ.at[pl.ds(s * RPT, RPT)])
    pltpu.sync_copy(zZ_hbm.at[pl.ds(0, CS)], ea_v)
    plsc.subcore_barrier()

    def body(j, carry):
        base = (wid * CH + j) * CS
        pltpu.sync_copy(dstb_hbm.at[pl.ds(base, CS)], dst_cur)
        pltpu.sync_copy(ea_hbm.at[pl.ds(base, CS)], ea32_v)

        def cp(r, carry2):
            ea_v[r, 0:16] = ea32_v[r, 0:16]
            ea_v[r, 16:32] = ea32_v[r, 16:32]
            return carry2

        lax.fori_loop(0, CS, cp, 0)
        pltpu.sync_copy(ea_v, accZ.at[dst_cur], add=True)
        return carry

    lax.fori_loop(0, nch, body, 0)
    plsc.subcore_barrier()
    pltpu.sync_copy(accZ.at[pl.ds(s * RPT, RPT)],
                    outZ_hbm.at[pl.ds(c * NP + s * RPT, RPT)])


def _sc_passz(*args):
    return pl.kernel(
        _sc_passz_body,
        out_type=jax.ShapeDtypeStruct((NC * NP, D), _f32),
        mesh=_mesh(),
        scratch_types=[
            pltpu.VMEM((CS,), jnp.int32),
            pltpu.VMEM((CS, DZ), _f32),
            pltpu.VMEM((CS, D), _f32),
            pltpu.VMEM_SHARED((NP, D), _f32),
        ],
    )(*args)


def _wprep_body(wm_ref, bm_ref, wl_ref, wa_ref, wz_ref):
    wl = wl_ref[...]
    wa_ref[...] = jnp.dot(wm_ref[0:D, :], wl, preferred_element_type=_f32)
    wz_ref[0:DE, :] = jnp.dot(wm_ref[D:D + DE, :], wl,
                              preferred_element_type=_f32)
    bmw = jnp.dot(bm_ref[...], wl, preferred_element_type=_f32)
    wz_ref[DE:DZ, :] = jnp.broadcast_to(bmw, (DZ - DE, D)) / (DZ - DE)
    wz_ref[DZ:D, :] = jnp.zeros((D - DZ, D), _f32)


def _wprep(wm, bm, wl):
    return pl.pallas_call(
        _wprep_body,
        out_shape=[jax.ShapeDtypeStruct((D, D), _f32),
                   jax.ShapeDtypeStruct((D, D), _f32)],
    )(wm, bm.reshape(1, D), wl)


def _dense_body(a0, a1, xin, z0, z1, wa, wr, wz, b, h_ref):
    a = a0[...] + a1[...]
    z = z0[...] + z1[...]
    acc = jnp.dot(a, wa[...], preferred_element_type=_f32)
    acc = acc + jnp.dot(xin[...], wr[...], preferred_element_type=_f32)
    acc = acc + jnp.dot(z, wz[...], preferred_element_type=_f32)
    h_ref[...] = jnp.maximum(acc + b[...], 0.0)


def _dense(A, xin, Z, wa, wr, wz, b):
    return pl.pallas_call(
        _dense_body,
        grid=(NBLK,),
        in_specs=[
            pl.BlockSpec((BLK, D), lambda i: (i, 0)),
            pl.BlockSpec((BLK, D), lambda i: (NBLK + i, 0)),
            pl.BlockSpec((BLK, D), lambda i: (i, 0)),
            pl.BlockSpec((BLK, D), lambda i: (i, 0)),
            pl.BlockSpec((BLK, D), lambda i: (NBLK + i, 0)),
            pl.BlockSpec((D, D), lambda i: (0, 0)),
            pl.BlockSpec((D, D), lambda i: (0, 0)),
            pl.BlockSpec((D, D), lambda i: (0, 0)),
            pl.BlockSpec((1, D), lambda i: (0, 0)),
        ],
        out_specs=pl.BlockSpec((BLK, D), lambda i: (i, 0)),
        out_shape=jax.ShapeDtypeStruct((NP, D), _f32),
    )(A, A, xin, Z, Z, wa, wr, wz, b.reshape(1, D))


def _dense2_body(a0, a1, xin, z0, z1, wa, wr, wz, b, bat, out_ref):
    i = pl.program_id(0)
    a = a0[...] + a1[...]
    z = z0[...] + z1[...]
    acc = jnp.dot(a, wa[...], preferred_element_type=_f32)
    acc = acc + jnp.dot(xin[...], wr[...], preferred_element_type=_f32)
    acc = acc + jnp.dot(z, wz[...], preferred_element_type=_f32)
    h2 = jnp.maximum(acc + b[...], 0.0)
    seg = bat[0]
    iota = lax.broadcasted_iota(jnp.int32, (NG, BLK), 0)
    oh = jnp.where(jnp.broadcast_to(seg, (NG, BLK)) == iota, 1.0, 0.0)
    contrib = jnp.dot(oh, h2, preferred_element_type=_f32)

    @pl.when(i == 0)
    def _():
        out_ref[...] = contrib

    @pl.when(i != 0)
    def _():
        out_ref[...] += contrib


def _dense2(A, h1, Z, wa, wr, wz, b, batch3):
    return pl.pallas_call(
        _dense2_body,
        grid=(NBLK,),
        in_specs=[
            pl.BlockSpec((BLK, D), lambda i: (i, 0)),
            pl.BlockSpec((BLK, D), lambda i: (NBLK + i, 0)),
            pl.BlockSpec((BLK, D), lambda i: (i, 0)),
            pl.BlockSpec((BLK, D), lambda i: (i, 0)),
            pl.BlockSpec((BLK, D), lambda i: (NBLK + i, 0)),
            pl.BlockSpec((D, D), lambda i: (0, 0)),
            pl.BlockSpec((D, D), lambda i: (0, 0)),
            pl.BlockSpec((D, D), lambda i: (0, 0)),
            pl.BlockSpec((1, D), lambda i: (0, 0)),
            pl.BlockSpec((1, 1, BLK), lambda i: (i, 0, 0)),
        ],
        out_specs=pl.BlockSpec((NG, D), lambda i: (0, 0)),
        out_shape=jax.ShapeDtypeStruct((NG, D), _f32),
    )(A, A, h1, Z, Z, wa, wr, wz, b.reshape(1, D), batch3)


def kernel(x, edge_index, edge_attr, batch,
           Wm0, bm0, Wl0, Wr0, b0,
           Wm1, bm1, Wl1, Wr1, b1):
    src = edge_index[0]
    dst = edge_index[1]
    pad_e = EP - E
    src_p = jnp.concatenate([src, jnp.zeros((pad_e,), jnp.int32)])
    dst_p = jnp.concatenate([dst, jnp.full((pad_e,), N, jnp.int32)])
    ea_p = jnp.concatenate(
        [edge_attr, jnp.ones((E, DZ - DE), _f32)], axis=1)
    ea_p = jnp.concatenate([ea_p, jnp.zeros((pad_e, DZ), _f32)], axis=0)
    x_p = jnp.concatenate([x, jnp.zeros((NP - N, D), _f32)], axis=0)
    batch3 = jnp.concatenate(
        [batch, jnp.full((NP - N,), NG, jnp.int32)]).reshape(NBLK, 1, BLK)
    zA = jnp.zeros((RPT, D), _f32)

    wa0, wz0 = _wprep(Wm0, bm0, Wl0)
    wa1, wz1 = _wprep(Wm1, bm1, Wl1)

    Zp = _sc_passz(ea_p, dst_p, zA)
    A0 = _sc_passa(x_p, src_p, dst_p, zA)
    h1 = _dense(A0, x_p, Zp, wa0, Wr0, wz0, b0)
    A1 = _sc_passa(h1, src_p, dst_p, zA)
    out = _dense2(A1, h1, Zp, wa1, Wr1, wz1, b1, batch3)
    return out


# speedup vs baseline: 2.6070x; 1.1093x over previous
import jax
import jax.numpy as jnp
from jax import lax
from jax.experimental import pallas as pl
from jax.experimental.pallas import tpu as pltpu
from jax.experimental.pallas import tpu_sc as plsc

N = 10000
E = 320000
D = 128
DE = 16
NG = 64
DZ = 32

NC = 2
NS = 16
NT = NC * NS
CS = 128
CH = 80
CSZ = 64
CHZ = 160
NBZ = 3
CSA = 120
CHA = 86
NBA = 2
EP = NT * CHA * CSA

NP = 10240
RPT = NP // NS
BLK = 512
NBLK = NP // BLK

_f32 = jnp.float32
import functools


@functools.lru_cache(maxsize=None)
def _mesh():
    return plsc.VectorSubcoreMesh(core_axis_name="c", subcore_axis_name="s",
                                  num_cores=NC, num_subcores=NS)


def _sc_passa_body(x_hbm, srcb_hbm, dstb_hbm, zA_hbm,
                   outA_hbm,
                   sb0, sb1, db0, db1, rb0, rb1,
                   i0, i1, g0, g1, t0, t1, accA):
    sbufs = (sb0, sb1)
    dbufs = (db0, db1)
    rbufs = (rb0, rb1)
    isems = (i0, i1)
    gsems = (g0, g1)
    ssems = (t0, t1)
    c = lax.axis_index("c")
    s = lax.axis_index("s")
    wid = s * NC + c
    ebase = wid * CHA * CSA
    nch = jnp.minimum(CHA, (E - ebase + CSA - 1) // CSA)
    ngrp = (nch + NBA - 1) // NBA
    pltpu.sync_copy(zA_hbm, accA.at[pl.ds(s * RPT, RPT)])
    plsc.subcore_barrier()

    def fetch_idx(j, b):
        pltpu.async_copy(srcb_hbm.at[pl.ds(ebase + j * CSA, CSA)],
                         sbufs[b], isems[b])
        pltpu.async_copy(dstb_hbm.at[pl.ds(ebase + j * CSA, CSA)],
                         dbufs[b], isems[b])

    def wait_idx(j, b):
        pltpu.make_async_copy(srcb_hbm.at[pl.ds(ebase + j * CSA, CSA)],
                              sbufs[b], isems[b]).wait()
        pltpu.make_async_copy(dstb_hbm.at[pl.ds(ebase + j * CSA, CSA)],
                              dbufs[b], isems[b]).wait()

    for b in range(NBA):
        @pl.when(b < nch)
        def _():
            fetch_idx(b, b)

    def group(g, carry):
        for b in range(NBA):
            j = g * NBA + b

            @pl.when(j < nch)
            def _():
                wait_idx(j, b)
                pltpu.async_copy(x_hbm.at[sbufs[b]], rbufs[b], gsems[b])
        for b in range(NBA):
            j = g * NBA + b

            @pl.when(j < nch)
            def _():
                pltpu.make_async_copy(x_hbm.at[sbufs[b]], rbufs[b],
                                      gsems[b]).wait()
                pltpu.async_copy(rbufs[b], accA.at[dbufs[b]], ssems[b],
                                 add=True)
        for b in range(NBA):
            j = g * NBA + b
            jn = j + NBA

            @pl.when(j < nch)
            def _():
                pltpu.make_async_copy(rbufs[b], accA.at[dbufs[b]],
                                      ssems[b]).wait()

                @pl.when(jn < nch)
                def _():
                    fetch_idx(jn, b)

        return carry

    lax.fori_loop(0, ngrp, group, 0)
    plsc.subcore_barrier()
    pltpu.sync_copy(accA.at[pl.ds(s * RPT, RPT)],
                    outA_hbm.at[pl.ds(c * NP + s * RPT, RPT)])


def _sc_passa(*args):
    return pl.kernel(
        _sc_passa_body,
        out_type=jax.ShapeDtypeStruct((NC * NP, D), _f32),
        mesh=_mesh(),
        scratch_types=[pltpu.VMEM((CSA,), jnp.int32) for _ in range(2 * NBA)]
          + [pltpu.VMEM((CSA, D), _f32) for _ in range(NBA)]
          + [pltpu.SemaphoreType.DMA for _ in range(3 * NBA)]
          + [pltpu.VMEM_SHARED((NP, D), _f32)],
    )(*args)


def _sc_passz_body(ea_hbm, dstb_hbm, zZ_hbm,
                   outZ_hbm,
                   db0, db1, db2, eb0, eb1, eb2, wb0, wb1, wb2,
                   i0, i1, i2, t0, t1, t2, accZ):
    dbufs = (db0, db1, db2)
    ebufs = (eb0, eb1, eb2)
    wbufs = (wb0, wb1, wb2)
    isems = (i0, i1, i2)
    ssems = (t0, t1, t2)
    c = lax.axis_index("c")
    s = lax.axis_index("s")
    wid = s * NC + c
    ebase = wid * CHZ * CSZ
    nch = jnp.minimum(CHZ, (E - ebase + CSZ - 1) // CSZ)
    ngrp = (nch + NBZ - 1) // NBZ
    pltpu.sync_copy(zZ_hbm, accZ.at[pl.ds(s * RPT, RPT)])
    for b in range(NBZ):
        pltpu.sync_copy(zZ_hbm.at[pl.ds(0, CSZ)], wbufs[b])
    plsc.subcore_barrier()

    def fetch(j, b):
        pltpu.async_copy(dstb_hbm.at[pl.ds(ebase + j * CSZ, CSZ)],
                         dbufs[b], isems[b])
        pltpu.async_copy(ea_hbm.at[pl.ds((ebase + j * CSZ) * DZ, CSZ * DZ)],
                         ebufs[b], isems[b])

    def wait_fetch(j, b):
        pltpu.make_async_copy(dstb_hbm.at[pl.ds(ebase + j * CSZ, CSZ)],
                              dbufs[b], isems[b]).wait()
        pltpu.make_async_copy(ea_hbm.at[pl.ds((ebase + j * CSZ) * DZ,
                                              CSZ * DZ)],
                              ebufs[b], isems[b]).wait()

    for b in range(NBZ):
        @pl.when(b < nch)
        def _():
            fetch(b, b)

    def group(g, carry):
        for b in range(NBZ):
            j = g * NBZ + b

            @pl.when(j < nch)
            def _():
                wait_fetch(j, b)

                def cp(r, carry2):
                    wbufs[b][r, 0:16] = ebufs[b][pl.ds(r * DZ, 16)]
                    wbufs[b][r, 16:32] = ebufs[b][pl.ds(r * DZ + 16, 16)]
                    return carry2

                lax.fori_loop(0, CSZ, cp, 0)
                pltpu.async_copy(wbufs[b], accZ.at[dbufs[b]], ssems[b],
                                 add=True)
        for b in range(NBZ):
            j = g * NBZ + b
            jn = j + NBZ

            @pl.when(j < nch)
            def _():
                pltpu.make_async_copy(wbufs[b], accZ.at[dbufs[b]],
                                      ssems[b]).wait()

                @pl.when(jn < nch)
                def _():
                    fetch(jn, b)

        return carry

    lax.fori_loop(0, ngrp, group, 0)
    plsc.subcore_barrier()
    pltpu.sync_copy(accZ.at[pl.ds(s * RPT, RPT)],
                    outZ_hbm.at[pl.ds(c * NP + s * RPT, RPT)])


def _sc_passz(*args):
    return pl.kernel(
        _sc_passz_body,
        out_type=jax.ShapeDtypeStruct((NC * NP, D), _f32),
        mesh=_mesh(),
        scratch_types=[pltpu.VMEM((CSZ,), jnp.int32) for _ in range(NBZ)]
          + [pltpu.VMEM((CSZ * DZ,), _f32) for _ in range(NBZ)]
          + [pltpu.VMEM((CSZ, D), _f32) for _ in range(NBZ)]
          + [pltpu.SemaphoreType.DMA for _ in range(2 * NBZ)]
          + [pltpu.VMEM_SHARED((NP, D), _f32)],
    )(*args)


def _wprep_body(wm_ref, bm_ref, wl_ref, wa_ref, wz_ref):
    wl = wl_ref[...]
    wa_ref[...] = jnp.dot(wm_ref[0:D, :], wl, preferred_element_type=_f32)
    wz_ref[0:DE, :] = jnp.dot(wm_ref[D:D + DE, :], wl,
                              preferred_element_type=_f32)
    bmw = jnp.dot(bm_ref[...], wl, preferred_element_type=_f32)
    wz_ref[DE:DZ, :] = jnp.broadcast_to(bmw, (DZ - DE, D)) / (DZ - DE)
    wz_ref[DZ:D, :] = jnp.zeros((D - DZ, D), _f32)


def _wprep(wm, bm, wl):
    return pl.pallas_call(
        _wprep_body,
        out_shape=[jax.ShapeDtypeStruct((D, D), _f32),
                   jax.ShapeDtypeStruct((D, D), _f32)],
    )(wm, bm.reshape(1, D), wl)


def _dense_body(a0, a1, xin, z0, z1, wa, wr, wz, b, h_ref):
    a = a0[...] + a1[...]
    z = z0[...] + z1[...]
    acc = jnp.dot(a, wa[...], preferred_element_type=_f32)
    acc = acc + jnp.dot(xin[...], wr[...], preferred_element_type=_f32)
    acc = acc + jnp.dot(z, wz[...], preferred_element_type=_f32)
    h_ref[...] = jnp.maximum(acc + b[...], 0.0)


def _dense(A, xin, Z, wa, wr, wz, b):
    return pl.pallas_call(
        _dense_body,
        grid=(NBLK,),
        in_specs=[
            pl.BlockSpec((BLK, D), lambda i: (i, 0)),
            pl.BlockSpec((BLK, D), lambda i: (NBLK + i, 0)),
            pl.BlockSpec((BLK, D), lambda i: (i, 0)),
            pl.BlockSpec((BLK, D), lambda i: (i, 0)),
            pl.BlockSpec((BLK, D), lambda i: (NBLK + i, 0)),
            pl.BlockSpec((D, D), lambda i: (0, 0)),
            pl.BlockSpec((D, D), lambda i: (0, 0)),
            pl.BlockSpec((D, D), lambda i: (0, 0)),
            pl.BlockSpec((1, D), lambda i: (0, 0)),
        ],
        out_specs=pl.BlockSpec((BLK, D), lambda i: (i, 0)),
        out_shape=jax.ShapeDtypeStruct((NP, D), _f32),
    )(A, A, xin, Z, Z, wa, wr, wz, b.reshape(1, D))


def _dense2_body(a0, a1, xin, z0, z1, wa, wr, wz, b, bat, out_ref):
    i = pl.program_id(0)
    a = a0[...] + a1[...]
    z = z0[...] + z1[...]
    acc = jnp.dot(a, wa[...], preferred_element_type=_f32)
    acc = acc + jnp.dot(xin[...], wr[...], preferred_element_type=_f32)
    acc = acc + jnp.dot(z, wz[...], preferred_element_type=_f32)
    h2 = jnp.maximum(acc + b[...], 0.0)
    seg = bat[0]
    iota = lax.broadcasted_iota(jnp.int32, (NG, BLK), 0)
    oh = jnp.where(jnp.broadcast_to(seg, (NG, BLK)) == iota, 1.0, 0.0)
    contrib = jnp.dot(oh, h2, preferred_element_type=_f32)

    @pl.when(i == 0)
    def _():
        out_ref[...] = contrib

    @pl.when(i != 0)
    def _():
        out_ref[...] += contrib


def _dense2(A, h1, Z, wa, wr, wz, b, batch3):
    return pl.pallas_call(
        _dense2_body,
        grid=(NBLK,),
        in_specs=[
            pl.BlockSpec((BLK, D), lambda i: (i, 0)),
            pl.BlockSpec((BLK, D), lambda i: (NBLK + i, 0)),
            pl.BlockSpec((BLK, D), lambda i: (i, 0)),
            pl.BlockSpec((BLK, D), lambda i: (i, 0)),
            pl.BlockSpec((BLK, D), lambda i: (NBLK + i, 0)),
            pl.BlockSpec((D, D), lambda i: (0, 0)),
            pl.BlockSpec((D, D), lambda i: (0, 0)),
            pl.BlockSpec((D, D), lambda i: (0, 0)),
            pl.BlockSpec((1, D), lambda i: (0, 0)),
            pl.BlockSpec((1, 1, BLK), lambda i: (i, 0, 0)),
        ],
        out_specs=pl.BlockSpec((NG, D), lambda i: (0, 0)),
        out_shape=jax.ShapeDtypeStruct((NG, D), _f32),
    )(A, A, h1, Z, Z, wa, wr, wz, b.reshape(1, D), batch3)


def kernel(x, edge_index, edge_attr, batch,
           Wm0, bm0, Wl0, Wr0, b0,
           Wm1, bm1, Wl1, Wr1, b1):
    src = edge_index[0]
    dst = edge_index[1]
    pad_e = EP - E
    src_p = jnp.concatenate([src, jnp.zeros((pad_e,), jnp.int32)])
    dst_p = jnp.concatenate([dst, jnp.full((pad_e,), N, jnp.int32)])
    ea_p = jnp.concatenate(
        [edge_attr, jnp.ones((E, DZ - DE), _f32)], axis=1)
    ea_p = jnp.concatenate([ea_p, jnp.zeros((pad_e, DZ), _f32)], axis=0)
    x_p = jnp.concatenate([x, jnp.zeros((NP - N, D), _f32)], axis=0)
    batch3 = jnp.concatenate(
        [batch, jnp.full((NP - N,), NG, jnp.int32)]).reshape(NBLK, 1, BLK)
    zA = jnp.zeros((RPT, D), _f32)

    wa0, wz0 = _wprep(Wm0, bm0, Wl0)
    wa1, wz1 = _wprep(Wm1, bm1, Wl1)

    Zp = _sc_passz(ea_p.reshape(-1), dst_p, zA)
    A0 = _sc_passa(x_p, src_p, dst_p, zA)
    h1 = _dense(A0, x_p, Zp, wa0, Wr0, wz0, b0)
    A1 = _sc_passa(h1, src_p, dst_p, zA)
    out = _dense2(A1, h1, Zp, wa1, Wr1, wz1, b1, batch3)
    return out
